# baseline (speedup 1.0000x reference)
import functools

import jax
import jax.numpy as jnp
import numpy as np
from jax.experimental import pallas as pl

N = 14
L = 128
NN = N * N
NNN = N * N * N
NP = 16
NPP = NP * NP
NR = N * NP * NP
PB = 2


def _dft_constants():
    n = np.arange(L)[:, None].astype(np.float64)
    k = np.arange(L)[None, :].astype(np.float64)
    ang = 2.0 * np.pi * n * k / L
    live_f = (k <= 64)
    fc = np.where(live_f, np.cos(ang), 0.0)
    fs = np.where((k >= 1) & (k <= 63), -np.sin(ang), 0.0)
    kk = np.arange(L)[:, None].astype(np.float64)
    ll = np.arange(L)[None, :].astype(np.float64)
    ang2 = 2.0 * np.pi * kk * ll / L
    w = np.where(kk == 0, 1.0, np.where(kk == 64, 1.0, np.where(kk <= 63, 2.0, 0.0)))
    gc = w * np.cos(ang2) / L
    gs = np.where((kk >= 1) & (kk <= 63), -2.0 * np.sin(ang2) / L, 0.0)
    pt = np.zeros((NN, NN))
    for a in range(N):
        for b in range(N):
            pt[a * N + b, b * N + a] = 1.0
    pr = np.zeros((L, L))
    for l in range(L):
        pr[(L - l) % L, l] = 1.0
    ppad = np.zeros((N * NP, NN))
    for b in range(N):
        for s in range(N):
            if s != b:
                ppad[b * NP + s, b * N + s] = 1.0
    fp = np.zeros((L, L))
    fp[:, :65] = fc[:, :65]
    fp[:, 65:] = fs[:, 1:64]
    gp = np.zeros((L, L))
    gp[:65, :] = gc[:65, :]
    gp[65:, :] = gs[1:64, :]
    mk = np.zeros((NR, L))
    for b in range(N):
        for a in range(NP):
            for c in range(NP):
                if a == b or c == b or a >= N or c >= N:
                    mk[b * NPP + a * NP + c, :] = 1.0
    pext = np.zeros((NN, NP * NP))
    for a in range(N):
        for c in range(N):
            pext[a * N + c, a * NP + c] = 1.0
    eye = np.eye(PB)
    pt = np.kron(eye, pt)
    ppad = np.kron(eye, ppad)
    pext = np.kron(eye, pext)
    mk = np.tile(mk, (PB, 1))
    ndg = np.ones((PB * NN, L))
    dg = np.zeros((PB * NN, L))
    for p in range(PB):
        for i in range(N):
            ndg[p * NN + i * (N + 1), :] = 0.0
            dg[p * NN + i * (N + 1), 0] = 1.0
    return (jnp.asarray(fp, jnp.float32), jnp.asarray(gp, jnp.float32),
            jnp.asarray(pt, jnp.float32), jnp.asarray(pr, jnp.float32),
            jnp.asarray(ppad, jnp.float32), jnp.asarray(pext, jnp.float32),
            jnp.asarray(mk, jnp.float32),
            jnp.asarray(ndg, jnp.float32), jnp.asarray(dg, jnp.float32))


def _body(inp_ref, st_ref, fp_ref, gp_ref, pt_ref, pr_ref,
          ppad_ref, pext_ref, mk_ref, ndg_ref, dg_ref, out_o_ref, out_s_ref):
    f32 = jnp.float32
    x0 = inp_ref[...]
    st = st_ref[...]
    ndg = ndg_ref[...]
    dg = dg_ref[...]
    z = x0 * st * ndg + dg

    dot = functools.partial(jax.lax.dot, preferred_element_type=f32)
    norm = lambda v: v * (1.0 / jnp.sum(v, axis=-1, keepdims=True))
    x = norm(z)
    xp = dot(ppad_ref[...], x)
    sp = dot(xp, fp_ref[...])
    lidx = jax.lax.broadcasted_iota(jnp.int32, (1, L), 1)
    spr = jnp.where((lidx == 0) | (lidx == 64), 0.0, jnp.roll(sp, 64, axis=-1))

    nb = PB * N
    sp1 = sp.reshape(nb, 1, NP, L)
    sp2 = sp.reshape(nb, NP, 1, L)
    spr1 = spr.reshape(nb, 1, NP, L)
    spr2 = spr.reshape(nb, NP, 1, L)
    av = sp1 * sp2
    ar = spr1 * spr2
    bv = sp1 * spr2
    br = spr1 * sp2
    pk = jnp.where(lidx <= 64, av + ar, bv - br)
    pk = pk.reshape(PB * NR, L)
    comb = dot(pk, gp_ref[...]) + mk_ref[...]

    comb3 = comb.reshape(nb, NPP, L)
    mps = []
    for p in range(PB):
        mp = comb3[p * N]
        for b in range(1, N):
            mp = mp * comb3[p * N + b]
        mps.append(mp)
    mp = jnp.concatenate(mps, axis=0)
    m = dot(pext_ref[...], mp)
    out_s_ref[...] = norm(m)

    xt = dot(pt_ref[...], x)
    rev = dot(xt, pr_ref[...])
    u = m * x * rev * ndg + dg
    out_o_ref[...] = norm(u)


def kernel(inputs, states):
    B = inputs.shape[0]
    fp, gp, pt, pr, ppad, pext, mk, ndg, dg = _dft_constants()
    full = lambda s: pl.BlockSpec(s, lambda i: (0, 0))
    blk = pl.BlockSpec((PB * NN, L), lambda i: (i, 0))
    out_o, out_s = pl.pallas_call(
        _body,
        grid=(B // PB,),
        in_specs=[blk, blk, full((L, L)), full((L, L)),
                  full((PB * NN, PB * NN)), full((L, L)),
                  full((PB * N * NP, PB * NN)), full((PB * NN, PB * NPP)),
                  full((PB * NR, L)), full((PB * NN, L)), full((PB * NN, L))],
        out_specs=[blk, blk],
        out_shape=[jax.ShapeDtypeStruct((B * NN, L), jnp.float32),
                   jax.ShapeDtypeStruct((B * NN, L), jnp.float32)],
    )(inputs.reshape(B * NN, L), states.reshape(B * NN, L),
      fp, gp, pt, pr, ppad, pext, mk, ndg, dg)
    return out_o.reshape(B, N, N, L), out_s.reshape(B, NN, L)


# speedup vs baseline: 1.0449x; 1.0449x over previous
import functools

import jax
import jax.numpy as jnp
import numpy as np
from jax.experimental import pallas as pl

N = 14
L = 128
NN = N * N
NNN = N * N * N
NP = 16
NPP = NP * NP
NR = N * NP * NP
PB = 2


def _dft_constants():
    n = np.arange(L)[:, None].astype(np.float64)
    k = np.arange(L)[None, :].astype(np.float64)
    ang = 2.0 * np.pi * n * k / L
    live_f = (k <= 64)
    fc = np.where(live_f, np.cos(ang), 0.0)
    fs = np.where((k >= 1) & (k <= 63), -np.sin(ang), 0.0)
    kk = np.arange(L)[:, None].astype(np.float64)
    ll = np.arange(L)[None, :].astype(np.float64)
    ang2 = 2.0 * np.pi * kk * ll / L
    w = np.where(kk == 0, 1.0, np.where(kk == 64, 1.0, np.where(kk <= 63, 2.0, 0.0)))
    gc = w * np.cos(ang2) / L
    gs = np.where((kk >= 1) & (kk <= 63), -2.0 * np.sin(ang2) / L, 0.0)
    pt = np.zeros((NN, NN))
    for a in range(N):
        for b in range(N):
            pt[a * N + b, b * N + a] = 1.0
    pr = np.zeros((L, L))
    for l in range(L):
        pr[(L - l) % L, l] = 1.0
    ppad = np.zeros((N * NP, NN))
    for b in range(N):
        for s in range(N):
            if s != b:
                ppad[b * NP + s, b * N + s] = 1.0
    fp = np.zeros((L, L))
    fp[:, :65] = fc[:, :65]
    fp[:, 65:] = fs[:, 1:64]
    gp = np.zeros((L, L))
    gp[:65, :] = gc[:65, :]
    gp[65:, :] = gs[1:64, :]
    mk = np.zeros((NR, L))
    for b in range(N):
        for a in range(NP):
            for c in range(NP):
                if a == b or c == b or a >= N or c >= N:
                    mk[b * NPP + a * NP + c, :] = 1.0
    pext = np.zeros((NN, NP * NP))
    for a in range(N):
        for c in range(N):
            pext[a * N + c, a * NP + c] = 1.0
    eye = np.eye(PB)
    pt = np.kron(eye, pt)
    ppad = np.kron(eye, ppad)
    pext = np.kron(eye, pext)
    mk = np.tile(mk, (PB, 1))
    return (jnp.asarray(fp, jnp.float32), jnp.asarray(gp, jnp.float32),
            jnp.asarray(pt, jnp.float32), jnp.asarray(pr, jnp.float32),
            jnp.asarray(ppad, jnp.float32), jnp.asarray(pext, jnp.float32),
            jnp.asarray(mk, jnp.float32))


def _body(inp_ref, st_ref, fp_ref, gp_ref, pt_ref, pr_ref,
          ppad_ref, pext_ref, mk_ref, out_o_ref, out_s_ref):
    f32 = jnp.float32
    x0 = inp_ref[...]
    st = st_ref[...]
    rows = jax.lax.broadcasted_iota(jnp.int32, (1, NN, 1), 1)
    lanes = jax.lax.broadcasted_iota(jnp.int32, (1, NN, L), 2)
    is_diag = (rows % (N + 1)) == 0
    dirac = jnp.where(lanes == 0, 1.0, 0.0).astype(f32)
    z = jnp.where(is_diag, dirac, x0 * st)

    dot = functools.partial(jax.lax.dot, preferred_element_type=f32)
    norm = lambda v: v * (1.0 / jnp.sum(v, axis=-1, keepdims=True))
    x = norm(z).reshape(PB * NN, L)
    xp = dot(ppad_ref[...], x)
    sp = dot(xp, fp_ref[...])
    lidx = jax.lax.broadcasted_iota(jnp.int32, (1, L), 1)
    spr = jnp.where((lidx == 0) | (lidx == 64), 0.0, jnp.roll(sp, 64, axis=-1))

    nb = PB * N
    sp1 = sp.reshape(nb, 1, NP, L)
    sp2 = sp.reshape(nb, NP, 1, L)
    spr1 = spr.reshape(nb, 1, NP, L)
    spr2 = spr.reshape(nb, NP, 1, L)
    av = sp1 * sp2
    ar = spr1 * spr2
    bv = sp1 * spr2
    br = spr1 * sp2
    pk = jnp.where(lidx <= 64, av + ar, bv - br)
    pk = pk.reshape(PB * NR, L)
    comb = dot(pk, gp_ref[...]) + mk_ref[...]

    comb3 = comb.reshape(nb, NPP, L)
    mps = []
    for p in range(PB):
        mp = comb3[p * N]
        for b in range(1, N):
            mp = mp * comb3[p * N + b]
        mps.append(mp)
    mp = jnp.concatenate(mps, axis=0)
    m = dot(pext_ref[...], mp)
    out_s_ref[...] = norm(m.reshape(PB, NN, L))

    xt = dot(pt_ref[...], x)
    rev = dot(xt, pr_ref[...])
    u = (m * x * rev).reshape(PB, NN, L)
    u = jnp.where(is_diag, dirac, u)
    out_o_ref[...] = norm(u)


def kernel(inputs, states):
    B = inputs.shape[0]
    fp, gp, pt, pr, ppad, pext, mk = _dft_constants()
    full = lambda s: pl.BlockSpec(s, lambda i: (0, 0))
    blk = pl.BlockSpec((PB, NN, L), lambda i: (i, 0, 0))
    out_o, out_s = pl.pallas_call(
        _body,
        grid=(B // PB,),
        in_specs=[blk, blk, full((L, L)), full((L, L)),
                  full((PB * NN, PB * NN)), full((L, L)),
                  full((PB * N * NP, PB * NN)), full((PB * NN, PB * NPP)),
                  full((PB * NR, L))],
        out_specs=[blk, blk],
        out_shape=[jax.ShapeDtypeStruct((B, NN, L), jnp.float32),
                   jax.ShapeDtypeStruct((B, NN, L), jnp.float32)],
    )(inputs, states, fp, gp, pt, pr, ppad, pext, mk)
    return out_o.reshape(B, N, N, L), out_s


# speedup vs baseline: 1.1203x; 1.0722x over previous
import functools

import jax
import jax.numpy as jnp
import numpy as np
from jax.experimental import pallas as pl

N = 14
L = 128
NN = N * N
NNN = N * N * N
NP = 16
NPP = NP * NP
NR = N * NP * NP
PB = 2


def _dft_constants():
    n = np.arange(L)[:, None].astype(np.float64)
    k = np.arange(L)[None, :].astype(np.float64)
    ang = 2.0 * np.pi * n * k / L
    live_f = (k <= 64)
    fc = np.where(live_f, np.cos(ang), 0.0)
    fs = np.where((k >= 1) & (k <= 63), -np.sin(ang), 0.0)
    kk = np.arange(L)[:, None].astype(np.float64)
    ll = np.arange(L)[None, :].astype(np.float64)
    ang2 = 2.0 * np.pi * kk * ll / L
    w = np.where(kk == 0, 1.0, np.where(kk == 64, 1.0, np.where(kk <= 63, 2.0, 0.0)))
    gc = w * np.cos(ang2) / L
    gs = np.where((kk >= 1) & (kk <= 63), -2.0 * np.sin(ang2) / L, 0.0)
    pt = np.zeros((NN, NN))
    for a in range(N):
        for b in range(N):
            pt[a * N + b, b * N + a] = 1.0
    pr = np.zeros((L, L))
    for l in range(L):
        pr[(L - l) % L, l] = 1.0
    ppad = np.zeros((N * NP, NN))
    for b in range(N):
        for s in range(N):
            if s != b:
                ppad[b * NP + s, b * N + s] = 1.0
    fp = np.zeros((L, L))
    fp[:, :65] = fc[:, :65]
    fp[:, 65:] = fs[:, 1:64]
    gp = np.zeros((L, L))
    gp[:65, :] = gc[:65, :]
    gp[65:, :] = gs[1:64, :]
    mk = np.zeros((NR, L))
    for b in range(N):
        for a in range(NP):
            for c in range(NP):
                if a == b or c == b or a >= N or c >= N:
                    mk[b * NPP + a * NP + c, :] = 1.0
    pext = np.zeros((NN, NP * NP))
    for a in range(N):
        for c in range(N):
            pext[a * N + c, a * NP + c] = 1.0
    eye = np.eye(PB)
    pt = np.kron(eye, pt)
    ppad = np.kron(eye, ppad)
    pext = np.kron(eye, pext)
    mk = np.tile(mk, (PB, 1))
    return (jnp.asarray(fp, jnp.float32), jnp.asarray(gp, jnp.float32),
            jnp.asarray(pt, jnp.float32), jnp.asarray(pr, jnp.float32),
            jnp.asarray(ppad, jnp.float32), jnp.asarray(pext, jnp.float32),
            jnp.asarray(mk, jnp.float32))


def _body(inp_ref, st_ref, fp_ref, gp_ref, pt_ref, pr_ref,
          ppad_ref, pext_ref, mk_ref, out_o_ref, out_s_ref):
    f32 = jnp.float32
    x0 = inp_ref[...]
    st = st_ref[...]
    rows = jax.lax.broadcasted_iota(jnp.int32, (1, NN, 1), 1)
    lanes = jax.lax.broadcasted_iota(jnp.int32, (1, NN, L), 2)
    is_diag = (rows % (N + 1)) == 0
    dirac = jnp.where(lanes == 0, 1.0, 0.0).astype(f32)
    z = jnp.where(is_diag, dirac, x0 * st)

    dot = functools.partial(jax.lax.dot, preferred_element_type=f32)
    norm = lambda v: v * (1.0 / jnp.sum(v, axis=-1, keepdims=True))
    x = norm(z).reshape(PB * NN, L)
    xp = dot(ppad_ref[...], x)
    sp = dot(xp, fp_ref[...])
    lidx = jax.lax.broadcasted_iota(jnp.int32, (1, L), 1)
    spr = jnp.where((lidx == 0) | (lidx == 64), 0.0, jnp.roll(sp, 64, axis=-1))

    nb = PB * N
    lo = lidx <= 64
    f2 = jnp.where(lo, sp, spr)
    g1 = jnp.where(lo, spr, -spr)
    g2 = jnp.where(lo, spr, sp)
    f1b = sp.reshape(nb, 1, NP, L)
    f2b = f2.reshape(nb, NP, 1, L)
    g1b = g1.reshape(nb, 1, NP, L)
    g2b = g2.reshape(nb, NP, 1, L)
    pk = (f1b * f2b + g1b * g2b).reshape(PB * NR, L)
    comb = dot(pk, gp_ref[...]) + mk_ref[...]

    comb3 = comb.reshape(nb, NPP, L)
    mps = []
    for p in range(PB):
        mp = comb3[p * N]
        for b in range(1, N):
            mp = mp * comb3[p * N + b]
        mps.append(mp)
    mp = jnp.concatenate(mps, axis=0)
    m = dot(pext_ref[...], mp)
    out_s_ref[...] = norm(m.reshape(PB, NN, L))

    xt = dot(pt_ref[...], x)
    rev = dot(xt, pr_ref[...])
    u = (m * x * rev).reshape(PB, NN, L)
    u = jnp.where(is_diag, dirac, u)
    out_o_ref[...] = norm(u)


def kernel(inputs, states):
    B = inputs.shape[0]
    fp, gp, pt, pr, ppad, pext, mk = _dft_constants()
    full = lambda s: pl.BlockSpec(s, lambda i: (0, 0))
    blk = pl.BlockSpec((PB, NN, L), lambda i: (i, 0, 0))
    out_o, out_s = pl.pallas_call(
        _body,
        grid=(B // PB,),
        in_specs=[blk, blk, full((L, L)), full((L, L)),
                  full((PB * NN, PB * NN)), full((L, L)),
                  full((PB * N * NP, PB * NN)), full((PB * NN, PB * NPP)),
                  full((PB * NR, L))],
        out_specs=[blk, blk],
        out_shape=[jax.ShapeDtypeStruct((B, NN, L), jnp.float32),
                   jax.ShapeDtypeStruct((B, NN, L), jnp.float32)],
    )(inputs, states, fp, gp, pt, pr, ppad, pext, mk)
    return out_o.reshape(B, N, N, L), out_s
